# baseline (speedup 1.0000x reference)
import functools

import jax
import jax.numpy as jnp
from jax import lax
from jax.experimental import pallas as pl
from jax.experimental.pallas import tpu as pltpu
from jax.experimental.pallas import tpu_sc as plsc

B = 1024
N = 100000
TOTAL = B * N
NUM_WORKERS = 32
PER_W = TOTAL // NUM_WORKERS
CHUNK = 32_000
NCHUNKS = PER_W // CHUNK
LANES = 16

_mesh = plsc.VectorSubcoreMesh(core_axis_name="c", subcore_axis_name="s")


@functools.partial(
    pl.kernel,
    mesh=_mesh,
    out_type=[
        jax.ShapeDtypeStruct((TOTAL,), jnp.float32),
        jax.ShapeDtypeStruct((TOTAL,), jnp.float32),
    ],
    scratch_types=[
        pltpu.VMEM((CHUNK,), jnp.float32),
        pltpu.VMEM((CHUNK,), jnp.float32),
    ],
)
def _trace_update(inp_hbm, x_hbm, s_hbm, xn_hbm, inp_v, x_v):
    wid = lax.axis_index("s") * 2 + lax.axis_index("c")
    wbase = wid * PER_W

    def chunk_body(g, carry):
        base = wbase + g * CHUNK
        pltpu.sync_copy(inp_hbm.at[pl.ds(base, CHUNK)], inp_v)
        pltpu.sync_copy(x_hbm.at[pl.ds(base, CHUNK)], x_v)

        def vec_body(i, c):
            off = i * LANES
            iv = inp_v[pl.ds(off, LANES)]
            xv = x_v[pl.ds(off, LANES)]
            decayed = xv - jnp.float32(0.05) * xv
            ones = jnp.full((LANES,), 1.0, jnp.float32)
            x_v[pl.ds(off, LANES)] = jnp.where(iv != 0.0, ones, decayed)
            return c

        lax.fori_loop(0, CHUNK // LANES, vec_body, 0)
        pltpu.sync_copy(inp_v, s_hbm.at[pl.ds(base, CHUNK)])
        pltpu.sync_copy(x_v, xn_hbm.at[pl.ds(base, CHUNK)])
        return carry

    lax.fori_loop(0, NCHUNKS, chunk_body, 0)


def kernel(inpts, x):
    s_f, xn_f = _trace_update(inpts.reshape(TOTAL), x.reshape(TOTAL))
    return s_f.reshape(B, N), xn_f.reshape(B, N)


# speedup vs baseline: 1.3519x; 1.3519x over previous
import functools

import jax
import jax.numpy as jnp
from jax import lax
from jax.experimental import pallas as pl
from jax.experimental.pallas import tpu as pltpu
from jax.experimental.pallas import tpu_sc as plsc

B = 1024
N = 100000
TOTAL = B * N
NUM_WORKERS = 32
PER_W = TOTAL // NUM_WORKERS
CHUNK = 16_000
NCHUNKS = PER_W // CHUNK
NBUF = 4
LANES = 16
UNROLL = 8

_mesh = plsc.VectorSubcoreMesh(core_axis_name="c", subcore_axis_name="s")

_scratch = (
    [pltpu.VMEM((CHUNK,), jnp.float32) for _ in range(2 * NBUF)]
    + [pltpu.SemaphoreType.DMA for _ in range(2 * NBUF)]
)


@functools.partial(
    pl.kernel,
    mesh=_mesh,
    out_type=[
        jax.ShapeDtypeStruct((TOTAL,), jnp.float32),
        jax.ShapeDtypeStruct((TOTAL,), jnp.float32),
    ],
    scratch_types=_scratch,
)
def _trace_update(inp_hbm, x_hbm, s_hbm, xn_hbm, *refs):
    inp_v = refs[0:NBUF]
    x_v = refs[NBUF:2 * NBUF]
    ld = refs[2 * NBUF:3 * NBUF]
    st = refs[3 * NBUF:4 * NBUF]

    wid = lax.axis_index("s") * 2 + lax.axis_index("c")
    wbase = wid * PER_W

    def start_load(c, b):
        base = wbase + c * CHUNK
        pltpu.async_copy(inp_hbm.at[pl.ds(base, CHUNK)], inp_v[b], ld[b])
        pltpu.async_copy(x_hbm.at[pl.ds(base, CHUNK)], x_v[b], ld[b])

    def wait_load(b):
        pltpu.make_async_copy(inp_hbm.at[pl.ds(0, CHUNK)], inp_v[b], ld[b]).wait()
        pltpu.make_async_copy(x_hbm.at[pl.ds(0, CHUNK)], x_v[b], ld[b]).wait()

    def start_store(c, b):
        base = wbase + c * CHUNK
        pltpu.async_copy(inp_v[b], s_hbm.at[pl.ds(base, CHUNK)], st[b])
        pltpu.async_copy(x_v[b], xn_hbm.at[pl.ds(base, CHUNK)], st[b])

    def wait_store(b):
        pltpu.make_async_copy(inp_v[b], s_hbm.at[pl.ds(0, CHUNK)], st[b]).wait()
        pltpu.make_async_copy(x_v[b], xn_hbm.at[pl.ds(0, CHUNK)], st[b]).wait()

    start_load(0, 0)
    start_load(1, 1)

    def quad_body(g, carry):
        for b in range(NBUF):
            c = g * NBUF + b
            br = (b + 2) % NBUF

            @pl.when(c >= 2)
            def _():
                wait_store(br)

            @pl.when(c + 2 < NCHUNKS)
            def _():
                start_load(c + 2, br)

            wait_load(b)

            def vec_body(i, cc):
                for u in range(UNROLL):
                    off = (i * UNROLL + u) * LANES
                    iv = inp_v[b][pl.ds(off, LANES)]
                    xv = x_v[b][pl.ds(off, LANES)]
                    decayed = xv - jnp.float32(0.05) * xv
                    ones = jnp.full((LANES,), 1.0, jnp.float32)
                    x_v[b][pl.ds(off, LANES)] = jnp.where(iv != 0.0, ones, decayed)
                return cc

            lax.fori_loop(0, CHUNK // (LANES * UNROLL), vec_body, 0)
            start_store(c, b)
        return carry

    lax.fori_loop(0, NCHUNKS // NBUF, quad_body, 0)

    wait_store((NCHUNKS - 2) % NBUF)
    wait_store((NCHUNKS - 1) % NBUF)


def kernel(inpts, x):
    s_f, xn_f = _trace_update(inpts.reshape(TOTAL), x.reshape(TOTAL))
    return s_f.reshape(B, N), xn_f.reshape(B, N)


# speedup vs baseline: 1.3527x; 1.0005x over previous
import functools

import jax
import jax.numpy as jnp
from jax import lax
from jax.experimental import pallas as pl
from jax.experimental.pallas import tpu as pltpu
from jax.experimental.pallas import tpu_sc as plsc

B = 1024
N = 100000
TOTAL = B * N
NUM_WORKERS = 32
PER_W = TOTAL // NUM_WORKERS
CHUNK = 16_000
NCHUNKS = PER_W // CHUNK
NBUF = 4
LANES = 16
UNROLL = 8

_mesh = plsc.VectorSubcoreMesh(core_axis_name="c", subcore_axis_name="s")

_scratch = (
    [pltpu.VMEM((CHUNK,), jnp.float32) for _ in range(2 * NBUF)]
    + [pltpu.SemaphoreType.DMA for _ in range(2 * NBUF)]
)


@functools.partial(
    pl.kernel,
    mesh=_mesh,
    out_type=[
        jax.ShapeDtypeStruct((TOTAL,), jnp.float32),
        jax.ShapeDtypeStruct((TOTAL,), jnp.float32),
    ],
    scratch_types=_scratch,
)
def _trace_update(inp_hbm, x_hbm, s_hbm, xn_hbm, *refs):
    inp_v = refs[0:NBUF]
    x_v = refs[NBUF:2 * NBUF]
    ld = refs[2 * NBUF:3 * NBUF]
    st = refs[3 * NBUF:4 * NBUF]

    wid = lax.axis_index("s") * 2 + lax.axis_index("c")
    wbase = wid * PER_W

    def start_load(c, b):
        base = wbase + c * CHUNK
        pltpu.async_copy(inp_hbm.at[pl.ds(base, CHUNK)], inp_v[b], ld[b])
        pltpu.async_copy(x_hbm.at[pl.ds(base, CHUNK)], x_v[b], ld[b])

    def wait_load(b):
        pltpu.make_async_copy(inp_hbm.at[pl.ds(0, CHUNK)], inp_v[b], ld[b]).wait()
        pltpu.make_async_copy(x_hbm.at[pl.ds(0, CHUNK)], x_v[b], ld[b]).wait()

    def start_store(c, b):
        base = wbase + c * CHUNK
        pltpu.async_copy(inp_v[b], s_hbm.at[pl.ds(base, CHUNK)], st[b])
        pltpu.async_copy(x_v[b], xn_hbm.at[pl.ds(base, CHUNK)], st[b])

    def wait_store(b):
        pltpu.make_async_copy(inp_v[b], s_hbm.at[pl.ds(0, CHUNK)], st[b]).wait()
        pltpu.make_async_copy(x_v[b], xn_hbm.at[pl.ds(0, CHUNK)], st[b]).wait()

    start_load(0, 0)
    start_load(1, 1)

    def quad_body(g, carry):
        for b in range(NBUF):
            c = g * NBUF + b
            br = (b + 2) % NBUF

            @pl.when(c >= 2)
            def _():
                wait_store(br)

            @pl.when(c + 2 < NCHUNKS)
            def _():
                start_load(c + 2, br)

            wait_load(b)

            @plsc.parallel_loop(0, CHUNK, step=LANES, unroll=UNROLL)
            def vec_body(off):
                iv = inp_v[b][pl.ds(off, LANES)]
                xv = x_v[b][pl.ds(off, LANES)]
                decayed = xv - jnp.float32(0.05) * xv
                ones = jnp.full((LANES,), 1.0, jnp.float32)
                x_v[b][pl.ds(off, LANES)] = jnp.where(iv != 0.0, ones, decayed)

            start_store(c, b)
        return carry

    lax.fori_loop(0, NCHUNKS // NBUF, quad_body, 0)

    wait_store((NCHUNKS - 2) % NBUF)
    wait_store((NCHUNKS - 1) % NBUF)


def kernel(inpts, x):
    s_f, xn_f = _trace_update(inpts.reshape(TOTAL), x.reshape(TOTAL))
    return s_f.reshape(B, N), xn_f.reshape(B, N)


# speedup vs baseline: 2.6061x; 1.9266x over previous
import functools

import jax
import jax.numpy as jnp
from jax import lax
from jax.experimental import pallas as pl
from jax.experimental.pallas import tpu as pltpu
from jax.experimental.pallas import tpu_sc as plsc

B = 1024
N = 100000
NUM_WORKERS = 32
RB_PER_W = 4
ROWS_PER_W = 8 * RB_PER_W
CW = 1408
NALIGNED = 99968
CPR = NALIGNED // CW
NCHUNKS = RB_PER_W * CPR
NTAIL = N - NALIGNED
NBUF = 4
LANES = 16
UNROLL = 2

_mesh = plsc.VectorSubcoreMesh(core_axis_name="c", subcore_axis_name="s")

_scratch = (
    [pltpu.VMEM((8, CW), jnp.float32) for _ in range(2 * NBUF)]
    + [pltpu.SemaphoreType.DMA for _ in range(2 * NBUF)]
)


@functools.partial(
    pl.kernel,
    mesh=_mesh,
    out_type=[
        jax.ShapeDtypeStruct((B, N), jnp.float32),
        jax.ShapeDtypeStruct((B, N), jnp.float32),
    ],
    scratch_types=_scratch,
)
def _trace_update(inp_hbm, x_hbm, s_hbm, xn_hbm, *refs):
    inp_v = refs[0:NBUF]
    x_v = refs[NBUF:2 * NBUF]
    ld = refs[2 * NBUF:3 * NBUF]
    st = refs[3 * NBUF:4 * NBUF]

    wid = lax.axis_index("s") * 2 + lax.axis_index("c")
    row0 = wid * ROWS_PER_W

    def rowcol(c):
        return row0 + (c // CPR) * 8, (c % CPR) * CW

    def start_load(c, b):
        r, c0 = rowcol(c)
        pltpu.async_copy(inp_hbm.at[pl.ds(r, 8), pl.ds(c0, CW)], inp_v[b], ld[b])
        pltpu.async_copy(x_hbm.at[pl.ds(r, 8), pl.ds(c0, CW)], x_v[b], ld[b])

    def wait_load(b):
        pltpu.make_async_copy(inp_hbm.at[pl.ds(0, 8), pl.ds(0, CW)], inp_v[b], ld[b]).wait()
        pltpu.make_async_copy(x_hbm.at[pl.ds(0, 8), pl.ds(0, CW)], x_v[b], ld[b]).wait()

    def start_store(c, b):
        r, c0 = rowcol(c)
        pltpu.async_copy(inp_v[b], s_hbm.at[pl.ds(r, 8), pl.ds(c0, CW)], st[b])
        pltpu.async_copy(x_v[b], xn_hbm.at[pl.ds(r, 8), pl.ds(c0, CW)], st[b])

    def wait_store(b):
        pltpu.make_async_copy(inp_v[b], s_hbm.at[pl.ds(0, 8), pl.ds(0, CW)], st[b]).wait()
        pltpu.make_async_copy(x_v[b], xn_hbm.at[pl.ds(0, 8), pl.ds(0, CW)], st[b]).wait()

    def update(iv, xv):
        decayed = xv - jnp.float32(0.05) * xv
        ones = jnp.full((LANES,), 1.0, jnp.float32)
        return jnp.where(iv != 0.0, ones, decayed)

    start_load(0, 0)
    start_load(1, 1)

    def quad_body(g, carry):
        for b in range(NBUF):
            c = g * NBUF + b
            br = (b + 2) % NBUF

            @pl.when(c >= 2)
            def _():
                wait_store(br)

            @pl.when(c + 2 < NCHUNKS)
            def _():
                start_load(c + 2, br)

            wait_load(b)

            @plsc.parallel_loop(0, CW, step=LANES * UNROLL)
            def vec_body(off):
                for u in range(UNROLL):
                    o = off + u * LANES
                    for r in range(8):
                        iv = inp_v[b].at[r][pl.ds(o, LANES)]
                        xv = x_v[b].at[r][pl.ds(o, LANES)]
                        x_v[b].at[r][pl.ds(o, LANES)] = update(iv, xv)

            start_store(c, b)
        return carry

    lax.fori_loop(0, NCHUNKS // NBUF, quad_body, 0)

    wait_store((NCHUNKS - 2) % NBUF)
    wait_store((NCHUNKS - 1) % NBUF)


def kernel(inpts, x):
    s_k, xn_k = _trace_update(inpts, x)
    tail_in = lax.slice(inpts, (0, NALIGNED), (B, N))
    tail_x = lax.slice(x, (0, NALIGNED), (B, N))
    tail_new = jnp.where(tail_in != 0.0, jnp.float32(1.0),
                         tail_x - jnp.float32(0.05) * tail_x)
    s = lax.dynamic_update_slice(s_k, tail_in, (0, NALIGNED))
    xn = lax.dynamic_update_slice(xn_k, tail_new, (0, NALIGNED))
    return s, xn


# speedup vs baseline: 8.9405x; 3.4306x over previous
import functools

import jax
import jax.numpy as jnp
from jax import lax
from jax.experimental import pallas as pl
from jax.experimental.pallas import tpu as pltpu
from jax.experimental.pallas import tpu_sc as plsc

B = 1024
N = 100000
NUM_WORKERS = 32
RB = 8
NBLOCKS = N // RB
BASE_CH = NBLOCKS // NUM_WORKERS
EXTRA_W = NBLOCKS % NUM_WORKERS
NITER = 396
NBUF = 4
LANES = 16
UNROLL = 2

_mesh = plsc.VectorSubcoreMesh(core_axis_name="c", subcore_axis_name="s")

_scratch = (
    [pltpu.VMEM((RB, B), jnp.float32) for _ in range(2 * NBUF)]
    + [pltpu.SemaphoreType.DMA for _ in range(2 * NBUF)]
)


@functools.partial(
    pl.kernel,
    mesh=_mesh,
    out_type=[
        jax.ShapeDtypeStruct((N, B), jnp.float32),
        jax.ShapeDtypeStruct((N, B), jnp.float32),
    ],
    scratch_types=_scratch,
)
def _trace_update(inp_hbm, x_hbm, s_hbm, xn_hbm, *refs):
    inp_v = refs[0:NBUF]
    x_v = refs[NBUF:2 * NBUF]
    ld = refs[2 * NBUF:3 * NBUF]
    st = refs[3 * NBUF:4 * NBUF]

    wid = lax.axis_index("s") * 2 + lax.axis_index("c")
    nch = BASE_CH + jnp.where(wid < EXTRA_W, 1, 0).astype(jnp.int32)

    def row0(k):
        return (wid + k * NUM_WORKERS) * RB

    def start_load(k, b):
        r = row0(k)
        pltpu.async_copy(inp_hbm.at[pl.ds(r, RB), :], inp_v[b], ld[b])
        pltpu.async_copy(x_hbm.at[pl.ds(r, RB), :], x_v[b], ld[b])

    def wait_load(b):
        pltpu.make_async_copy(inp_hbm.at[pl.ds(0, RB), :], inp_v[b], ld[b]).wait()
        pltpu.make_async_copy(x_hbm.at[pl.ds(0, RB), :], x_v[b], ld[b]).wait()

    def start_store(k, b):
        r = row0(k)
        pltpu.async_copy(inp_v[b], s_hbm.at[pl.ds(r, RB), :], st[b])
        pltpu.async_copy(x_v[b], xn_hbm.at[pl.ds(r, RB), :], st[b])

    def wait_store(b):
        pltpu.make_async_copy(inp_v[b], s_hbm.at[pl.ds(0, RB), :], st[b]).wait()
        pltpu.make_async_copy(x_v[b], xn_hbm.at[pl.ds(0, RB), :], st[b]).wait()

    start_load(0, 0)
    start_load(1, 1)

    def quad_body(g, carry):
        for b in range(NBUF):
            k = g * NBUF + b
            br = (b + 2) % NBUF

            @pl.when((k >= 2) & (k - 2 < nch))
            def _():
                wait_store(br)

            @pl.when(k + 2 < nch)
            def _():
                start_load(k + 2, br)

            @pl.when(k < nch)
            def _():
                wait_load(b)

                @plsc.parallel_loop(0, B, step=LANES * UNROLL)
                def vec_body(off):
                    for u in range(UNROLL):
                        o = off + u * LANES
                        for r in range(RB):
                            iv = inp_v[b].at[r][pl.ds(o, LANES)]
                            xv = x_v[b].at[r][pl.ds(o, LANES)]
                            decayed = xv - jnp.float32(0.05) * xv
                            ones = jnp.full((LANES,), 1.0, jnp.float32)
                            x_v[b].at[r][pl.ds(o, LANES)] = jnp.where(
                                iv != 0.0, ones, decayed)

                start_store(k, b)
        return carry

    lax.fori_loop(0, NITER // NBUF, quad_body, 0)


def kernel(inpts, x):
    s_t, xn_t = _trace_update(inpts.T, x.T)
    return s_t.T, xn_t.T
